# baseline (speedup 1.0000x reference)
import jax
import jax.numpy as jnp
import numpy as np
from jax.experimental import pallas as pl

_RADIUS = np.float32(1.5 * 6 * 0.025)


def _bn(x, g, b):
    m = jnp.mean(x, axis=0)
    v = jnp.var(x, axis=0)
    return (x - m) / jnp.sqrt(v + 1e-5) * g + b


def _cconv(K, b, feats, pos_src, pos_dst, edges):
    src = edges[0]
    dst = edges[1]
    n_dst = pos_dst.shape[0]
    off = (pos_src[src] - pos_dst[dst]) / _RADIUS
    r2 = jnp.sum(off * off, axis=-1)
    win = jnp.clip((1.0 - r2) ** 3, 0.0, 1.0)
    u = jnp.clip((off + 1.0) * 1.5, 0.0, 3.0)
    i0f = jnp.clip(jnp.floor(u), 0.0, 2.0)
    f = u - i0f
    i0 = i0f.astype(jnp.int32)
    x_e = feats[src] * win[:, None]
    in_ch = feats.shape[1]
    acc = jnp.zeros((n_dst * 64, in_ch), feats.dtype)
    for dx in (0, 1):
        wx = f[:, 0] if dx == 1 else 1.0 - f[:, 0]
        for dy in (0, 1):
            wy = f[:, 1] if dy == 1 else 1.0 - f[:, 1]
            for dz in (0, 1):
                wz = f[:, 2] if dz == 1 else 1.0 - f[:, 2]
                w = wx * wy * wz
                cell = (i0[:, 0] + dx) * 16 + (i0[:, 1] + dy) * 4 + (i0[:, 2] + dz)
                acc = acc.at[dst * 64 + cell].add(x_e * w[:, None])
    acc = acc.reshape(n_dst, 64, in_ch)
    out_ch = K.shape[-1]
    return jnp.einsum('nki,kio->no', acc, K.reshape(64, in_ch, out_ch)) + b


def _iaff(x, y, pos, edges, p):
    xa = jnp.concatenate([x, y], axis=-1)
    xl = _cconv(p['iaff_K1'], p['iaff_b1'], xa, pos, pos, edges)
    xl = jax.nn.relu(_bn(xl, p['iaff_g1'], p['iaff_be1']))
    xl = _bn(_cconv(p['iaff_K2'], p['iaff_b2'], xl, pos, pos, edges), p['iaff_g2'], p['iaff_be2'])
    w1 = jax.nn.sigmoid(xl)
    xo = 2.0 * x * w1 + 2.0 * y * (1.0 - w1)
    xo2 = _cconv(p['iaff_K3'], p['iaff_b3'], xo, pos, pos, edges)
    xo2 = jax.nn.relu(_bn(xo2, p['iaff_g3'], p['iaff_be3']))
    xo2 = _bn(_cconv(p['iaff_K4'], p['iaff_b4'], xo2, pos, pos, edges), p['iaff_g4'], p['iaff_be4'])
    w2 = jax.nn.sigmoid(xo2)
    return 2.0 * x * w2 + 2.0 * y * (1.0 - w2)


def _scale_kernel(x_ref, o_ref):
    o_ref[...] = x_ref[...] * (1.0 / 128.0)


def kernel(pos, vel, box, box_feats, edge_fluid, edge_obs, params):
    p = params
    n = pos.shape[0]
    feats = jnp.concatenate([jnp.ones((n, 1), pos.dtype), vel], axis=-1)
    xf = _cconv(p['K0f'], p['b0f'], feats, pos, pos, edge_fluid)
    xo = _cconv(p['K0o'], p['b0o'], box_feats, box, pos, edge_obs)
    xd = feats @ p['W0'] + p['d0']
    x32 = _iaff(xf + xo, xd, pos, edge_fluid, p)
    x = jnp.concatenate([x32, xd], axis=-1)
    for i in range(1, 5):
        inp = x if i == 1 else jax.nn.relu(x)
        conv = _cconv(p['K%d' % i], p['bk%d' % i], inp, pos, pos, edge_fluid)
        dense = inp @ p['W%d' % i] + p['d%d' % i]
        y = conv + dense
        if y.shape[-1] == x.shape[-1]:
            y = y + x
        x = y
    return pl.pallas_call(
        _scale_kernel,
        out_shape=jax.ShapeDtypeStruct(x.shape, x.dtype),
    )(x)


# speedup vs baseline: 1.4583x; 1.4583x over previous
import functools

import jax
import jax.numpy as jnp
import numpy as np
from jax import lax
from jax.experimental import pallas as pl
from jax.experimental.pallas import tpu as pltpu
from jax.experimental.pallas import tpu_sc as plsc

_RADIUS = np.float32(1.5 * 6 * 0.025)
_INVR = np.float32(1.0 / _RADIUS)
_PAD_DST = np.int32(1 << 30)
_L = 16
_CE = 64
_SCAN = 128
_ROWS = 10240


def _ceil_to(x, m):
    return (x + m - 1) // m * m


def _make_geometry(n_src, n_dst, e_pad):
    W = e_pad // 32
    assert W % _SCAN == 0
    mesh = plsc.VectorSubcoreMesh(core_axis_name="c", subcore_axis_name="s")

    @functools.partial(
        pl.kernel,
        out_type=jax.ShapeDtypeStruct((e_pad, 16), jnp.float32),
        mesh=mesh,
        compiler_params=pltpu.CompilerParams(needs_layout_passes=False, use_tc_tiling_on_sc=False),
        scratch_types=[
            pltpu.VMEM((_SCAN,), jnp.int32),
            pltpu.VMEM((_SCAN,), jnp.int32),
            pltpu.VMEM((_SCAN,), jnp.int32),
            pltpu.VMEM((_SCAN, 16), jnp.float32),
            pltpu.VMEM((_SCAN, 16), jnp.float32),
            pltpu.VMEM((_SCAN, 16), jnp.float32),
        ],
    )
    def geom(ps_hbm, pd_hbm, src_hbm, dst_hbm, rec_hbm,
             srcb, dstb, dcb, psb, pdb, recb):
        c = lax.axis_index("c")
        s = lax.axis_index("s")
        wid = s * 2 + c
        base = wid * W
        lane = lax.iota(jnp.int32, 16)
        mx = ((lane >> 2) & 1) == 1
        my = ((lane >> 1) & 1) == 1
        mz = (lane & 1) == 1
        lt8 = lane < 8

        def chunk(k, _):
            off0 = base + k * _SCAN
            pltpu.sync_copy(src_hbm.at[pl.ds(off0, _SCAN)], srcb)
            pltpu.sync_copy(dst_hbm.at[pl.ds(off0, _SCAN)], dstb)
            for v in range(_SCAN // _L):
                dv = dstb[pl.ds(v * _L, _L)]
                dcb[pl.ds(v * _L, _L)] = jnp.minimum(dv, n_dst - 1)
            pltpu.sync_copy(ps_hbm.at[srcb], psb)
            pltpu.sync_copy(pd_hbm.at[dcb], pdb)

            def group(v, _):
                dv = dstb[pl.ds(v * _L, _L)]
                sv = srcb[pl.ds(v * _L, _L)]

                def edge(l, _):
                    j = v * _L + l
                    ps = psb[j]
                    pd = pdb[j]
                    off = (ps - pd) * _INVR
                    r2 = jnp.sum(off * off)
                    t = 1.0 - r2
                    win = jnp.clip(t * t * t, 0.0, 1.0)
                    u = jnp.clip((off + 1.0) * 1.5, 0.0, 3.0)
                    i0 = jnp.minimum(u.astype(jnp.int32), 2)
                    f = u - i0.astype(jnp.float32)
                    fx = f[0]
                    fy = f[1]
                    fz = f[2]
                    cell0 = i0[0] * 16 + i0[1] * 4 + i0[2]
                    wx = jnp.where(mx, fx, 1.0 - fx)
                    wy = jnp.where(my, fy, 1.0 - fy)
                    wz = jnp.where(mz, fz, 1.0 - fz)
                    dr = jnp.sum(jnp.where(lane == l, dv, 0))
                    wine = jnp.where(dr < n_dst, win, 0.0)
                    w8 = jnp.where(lt8, wx * wy * wz * wine, 0.0)
                    srcj = jnp.sum(jnp.where(lane == l, sv, 0))
                    ivec = (jnp.where(lane == 8, srcj, 0)
                            + jnp.where(lane == 9, dr, 0)
                            + jnp.where(lane == 10, cell0, 0))
                    recb[j] = jnp.where(lt8, w8,
                                        plsc.bitcast(ivec, jnp.float32))
                    return 0

                lax.fori_loop(0, _L, edge, 0)
                return 0

            lax.fori_loop(0, _SCAN // _L, group, 0)
            pltpu.sync_copy(recb, rec_hbm.at[pl.ds(off0, _SCAN)])
            return 0

        lax.fori_loop(0, W // _SCAN, chunk, 0)

    return geom


def _make_scatter(e_pad, n_src, in_p, td):
    ce = 32 if in_p >= 128 else _CE
    nt = _ROWS // td
    nt2 = nt // 2
    td_rows = td * 64
    esl = e_pad // 16
    assert esl % _SCAN == 0
    fr = td_rows // 16
    zr = 8192 // in_p
    nzc = fr // zr
    nv = in_p // _L
    sent = np.int32(e_pad - 1)
    mesh = plsc.VectorSubcoreMesh(core_axis_name="c", subcore_axis_name="s")

    @functools.partial(
        pl.kernel,
        out_type=jax.ShapeDtypeStruct((_ROWS * 64, in_p), jnp.float32),
        mesh=mesh,
        compiler_params=pltpu.CompilerParams(needs_layout_passes=False, use_tc_tiling_on_sc=False),
        scratch_types=[
            pltpu.VMEM_SHARED((td_rows, in_p), jnp.float32),
            pltpu.VMEM((esl + 64,), jnp.int32),
            pltpu.VMEM((_SCAN,), jnp.int32),
            pltpu.VMEM((ce, 16), jnp.float32),
            pltpu.VMEM((ce,), jnp.int32),
            pltpu.VMEM((ce, in_p), jnp.float32),
            pltpu.VMEM((8, ce, in_p), jnp.float32),
            pltpu.VMEM((8, ce), jnp.int32),
            pltpu.VMEM((zr, in_p), jnp.float32),
            pltpu.SemaphoreType.DMA,
        ],
    )
    def scat(rec_hbm, dst_hbm, ft_hbm, acc_hbm,
             acc_sh, eids, dstb, recb, srcb, featb, valb, idxb, zbuf, sem):
        c = lax.axis_index("c")
        s = lax.axis_index("s")
        lane = lax.iota(jnp.int32, 16)
        lt8 = lane < 8
        corner = (((lane >> 2) & 1) * 16 + ((lane >> 1) & 1) * 4 + (lane & 1))
        zv = jnp.zeros((16,), jnp.float32)

        def zrow(r, _):
            def zq(q, _):
                zbuf[r, pl.ds(q * _L, _L)] = zv
                return 0
            lax.fori_loop(0, nv, zq, 0)
            return 0

        lax.fori_loop(0, zr, zrow, 0)
        sbase = s * esl

        def tile(i, _):
            t = i * 2 + c
            lo = t * td

            def zc(z, _):
                pltpu.sync_copy(zbuf, acc_sh.at[pl.ds(s * fr + z * zr, zr)])
                return 0

            lax.fori_loop(0, nzc, zc, 0)
            plsc.subcore_barrier()

            def scan(k, cnt):
                pltpu.sync_copy(dst_hbm.at[pl.ds(sbase + k * _SCAN, _SCAN)],
                                dstb)
                for v in range(_SCAN // _L):
                    dv = dstb[pl.ds(v * _L, _L)]
                    m = (dv >= lo) & (dv < lo + td)
                    ev = sbase + k * _SCAN + v * _L + lane
                    plsc.store_compressed(eids.at[pl.ds(cnt, _L)], ev, mask=m)
                    cnt = cnt + plsc.all_reduce_population_count(m)[0]
                return cnt

            cnt = lax.fori_loop(0, esl // _SCAN, scan, jnp.int32(0))
            sv = jnp.full((16,), sent, jnp.int32)
            for v in range(4):
                eids[pl.ds(cnt + v * _L, _L)] = sv

            def chunk(ch, _):
                pltpu.sync_copy(rec_hbm.at[eids.at[pl.ds(ch * ce, ce)]],
                                recb)
                for v in range(ce // _L):
                    rowi = v * _L + lane
                    coli = jnp.full((16,), 8, jnp.int32)
                    sw = plsc.load_gather(recb, [rowi, coli])
                    srcb[pl.ds(v * _L, _L)] = plsc.bitcast(sw, jnp.int32)
                pltpu.sync_copy(ft_hbm.at[srcb], featb)

                def edge(j, _):
                    rec = recb[j]
                    ivec = plsc.bitcast(rec, jnp.int32)
                    dstj = ivec[9]
                    cell0 = ivec[10]
                    rows8 = (dstj - lo) * 64 + cell0 + corner
                    rows8 = jnp.clip(rows8, 0, td_rows - 1)
                    plsc.store_scatter(
                        idxb, [lane, jnp.full((16,), j, jnp.int32)],
                        rows8, mask=lt8)
                    vq = [featb[j, pl.ds(q * _L, _L)] for q in range(nv)]
                    for cc in range(8):
                        wc = rec[cc]
                        for q in range(nv):
                            valb[cc, j, pl.ds(q * _L, _L)] = wc * vq[q]
                    return 0

                lax.fori_loop(0, ce, edge, 0)
                hs = [pltpu.async_copy(valb.at[cc],
                                       acc_sh.at[idxb.at[cc]],
                                       sem, add=True)
                      for cc in range(8)]
                for h in hs:
                    h.wait()
                return 0

            lax.fori_loop(0, (cnt + ce - 1) // ce, chunk, 0)
            plsc.subcore_barrier()
            pltpu.sync_copy(acc_sh.at[pl.ds(s * fr, fr)],
                            acc_hbm.at[pl.ds(t * td_rows + s * fr, fr)])
            return 0

        lax.fori_loop(0, nt2, tile, 0)

    return scat


def _mm_kernel(x_ref, w_ref, b_ref, o_ref):
    o_ref[...] = (jnp.dot(x_ref[...], w_ref[...],
                          preferred_element_type=jnp.float32)
                  + b_ref[...])


def _mm_dense_kernel(x_ref, w_ref, y_ref, wd_ref, b_ref, o_ref):
    o_ref[...] = (jnp.dot(x_ref[...], w_ref[...],
                          preferred_element_type=jnp.float32)
                  + jnp.dot(y_ref[...], wd_ref[...],
                            preferred_element_type=jnp.float32)
                  + b_ref[...])


_BR = 512


def _tc_matmul(x, w, b, y=None, wd=None):
    kd = x.shape[1]
    b2 = jnp.broadcast_to(b[None, :], (_BR, 128))
    grid = (_ROWS // _BR,)
    if y is None:
        return pl.pallas_call(
            _mm_kernel,
            grid=grid,
            in_specs=[
                pl.BlockSpec((_BR, kd), lambda i: (i, 0)),
                pl.BlockSpec((kd, 128), lambda i: (0, 0)),
                pl.BlockSpec((_BR, 128), lambda i: (0, 0)),
            ],
            out_specs=pl.BlockSpec((_BR, 128), lambda i: (i, 0)),
            out_shape=jax.ShapeDtypeStruct((_ROWS, 128), jnp.float32),
        )(x, w, b2)
    ip = y.shape[1]
    return pl.pallas_call(
        _mm_dense_kernel,
        grid=grid,
        in_specs=[
            pl.BlockSpec((_BR, kd), lambda i: (i, 0)),
            pl.BlockSpec((kd, 128), lambda i: (0, 0)),
            pl.BlockSpec((_BR, ip), lambda i: (i, 0)),
            pl.BlockSpec((ip, 128), lambda i: (0, 0)),
            pl.BlockSpec((_BR, 128), lambda i: (0, 0)),
        ],
        out_specs=pl.BlockSpec((_BR, 128), lambda i: (i, 0)),
        out_shape=jax.ShapeDtypeStruct((_ROWS, 128), jnp.float32),
    )(x, w, y, wd, b2)


def _pad_cols(a, w):
    return jnp.pad(a, ((0, 0), (0, w - a.shape[1])))


def _kmat(K, in_p):
    ci, co = K.shape[3], K.shape[4]
    km = K.reshape(64, ci, co)
    km = jnp.pad(km, ((0, 0), (0, in_p - ci), (0, 128 - co)))
    return km.reshape(64 * in_p, 128)


_TD_BY_INP = {16: 640, 32: 320, 64: 160, 128: 80}


def _bn(x, g, b):
    m = jnp.mean(x, axis=0)
    v = jnp.var(x, axis=0)
    return (x - m) / jnp.sqrt(v + 1e-5) * g + b


def kernel(pos, vel, box, box_feats, edge_fluid, edge_obs, params):
    p = params
    n = pos.shape[0]
    m = box.shape[0]
    e = edge_fluid.shape[1]
    eo = edge_obs.shape[1]
    e_pad = _ceil_to(e, 4096)
    eo_pad = _ceil_to(eo, 4096)

    pos16 = _pad_cols(pos, 16)
    box16 = _pad_cols(box, 16)

    def pad_edges(edges, epad):
        src = jnp.pad(edges[0], (0, epad - edges.shape[1]))
        dst = jnp.pad(edges[1], (0, epad - edges.shape[1]),
                      constant_values=_PAD_DST)
        return src, dst

    src_f, dst_f = pad_edges(edge_fluid, e_pad)
    src_o, dst_o = pad_edges(edge_obs, eo_pad)

    rec_f = _make_geometry(n, n, e_pad)(pos16, pos16, src_f, dst_f)
    rec_o = _make_geometry(m, n, eo_pad)(box16, pos16, src_o, dst_o)

    scat_cache = {}

    def conv(K, b, feats, rec, dst, epad, y=None, wd=None):
        ci = K.shape[3]
        in_p = max(16, _ceil_to(ci, 16))
        td = _TD_BY_INP[in_p]
        key = (epad, in_p)
        if key not in scat_cache:
            scat_cache[key] = _make_scatter(epad, feats.shape[0], in_p, td)
        ftp = _pad_cols(feats, in_p)
        acc = scat_cache[key](rec, dst, ftp)
        acc = acc.reshape(_ROWS, 64 * in_p)
        km = _kmat(K, in_p)
        if y is not None:
            ip = y.shape[1]
            yp = jnp.pad(y, ((0, _ROWS - n), (0, 0)))
            wdp = jnp.pad(wd, ((0, 0), (0, 128 - wd.shape[1])))
            out = _tc_matmul(acc, km, jnp.pad(b, (0, 128 - b.shape[0])),
                             yp, wdp)
        else:
            out = _tc_matmul(acc, km, jnp.pad(b, (0, 128 - b.shape[0])))
        return out[:n, :K.shape[4]]

    def fconv(K, b, feats, y=None, wd=None):
        return conv(K, b, feats, rec_f, dst_f, e_pad, y=y, wd=wd)

    feats = jnp.concatenate([jnp.ones((n, 1), pos.dtype), vel], axis=-1)
    xf = fconv(p['K0f'], p['b0f'], feats)
    xo = conv(p['K0o'], p['b0o'], box_feats, rec_o, dst_o, eo_pad)
    xd = feats @ p['W0'] + p['d0']

    x, y = xf + xo, xd
    xa = jnp.concatenate([x, y], axis=-1)
    xl = fconv(p['iaff_K1'], p['iaff_b1'], xa)
    xl = jax.nn.relu(_bn(xl, p['iaff_g1'], p['iaff_be1']))
    xl = _bn(fconv(p['iaff_K2'], p['iaff_b2'], xl), p['iaff_g2'], p['iaff_be2'])
    w1 = jax.nn.sigmoid(xl)
    xg = 2.0 * x * w1 + 2.0 * y * (1.0 - w1)
    t = fconv(p['iaff_K3'], p['iaff_b3'], xg)
    t = jax.nn.relu(_bn(t, p['iaff_g3'], p['iaff_be3']))
    t = _bn(fconv(p['iaff_K4'], p['iaff_b4'], t), p['iaff_g4'], p['iaff_be4'])
    w2 = jax.nn.sigmoid(t)
    x32 = 2.0 * x * w2 + 2.0 * y * (1.0 - w2)

    x = jnp.concatenate([x32, xd], axis=-1)
    for i in range(1, 5):
        inp = x if i == 1 else jax.nn.relu(x)
        y = fconv(p['K%d' % i], p['bk%d' % i], inp,
                  y=inp, wd=p['W%d' % i]) + p['d%d' % i]
        if y.shape[-1] == x.shape[-1]:
            y = y + x
        x = y
    return x * (1.0 / 128.0)


# speedup vs baseline: 2.3985x; 1.6447x over previous
import functools

import jax
import jax.numpy as jnp
import numpy as np
from jax import lax
from jax.experimental import pallas as pl
from jax.experimental.pallas import tpu as pltpu
from jax.experimental.pallas import tpu_sc as plsc

_RADIUS = np.float32(1.5 * 6 * 0.025)
_INVR = np.float32(1.0 / _RADIUS)
_PAD_DST = np.int32(1 << 30)
_L = 16
_CE = 64
_SCAN = 128
_ROWS = 10240


def _ceil_to(x, m):
    return (x + m - 1) // m * m


def _make_geometry(n_src, n_dst, e_pad):
    W = e_pad // 32
    assert W % _SCAN == 0
    mesh = plsc.VectorSubcoreMesh(core_axis_name="c", subcore_axis_name="s")

    @functools.partial(
        pl.kernel,
        out_type=jax.ShapeDtypeStruct((e_pad, 16), jnp.float32),
        mesh=mesh,
        compiler_params=pltpu.CompilerParams(needs_layout_passes=False, use_tc_tiling_on_sc=False),
        scratch_types=[
            pltpu.VMEM((_SCAN,), jnp.int32),
            pltpu.VMEM((_SCAN,), jnp.int32),
            pltpu.VMEM((_SCAN,), jnp.int32),
            pltpu.VMEM((_SCAN, 16), jnp.float32),
            pltpu.VMEM((_SCAN, 16), jnp.float32),
            pltpu.VMEM((_SCAN, 16), jnp.float32),
        ],
    )
    def geom(ps_hbm, pd_hbm, src_hbm, dst_hbm, rec_hbm,
             srcb, dstb, dcb, psb, pdb, recb):
        c = lax.axis_index("c")
        s = lax.axis_index("s")
        wid = s * 2 + c
        base = wid * W
        lane = lax.iota(jnp.int32, 16)
        mx = ((lane >> 2) & 1) == 1
        my = ((lane >> 1) & 1) == 1
        mz = (lane & 1) == 1
        lt8 = lane < 8

        def chunk(k, _):
            off0 = base + k * _SCAN
            pltpu.sync_copy(src_hbm.at[pl.ds(off0, _SCAN)], srcb)
            pltpu.sync_copy(dst_hbm.at[pl.ds(off0, _SCAN)], dstb)
            for v in range(_SCAN // _L):
                dv = dstb[pl.ds(v * _L, _L)]
                dcb[pl.ds(v * _L, _L)] = jnp.minimum(dv, n_dst - 1)
            pltpu.sync_copy(ps_hbm.at[srcb], psb)
            pltpu.sync_copy(pd_hbm.at[dcb], pdb)

            def group(v, _):
                dv = dstb[pl.ds(v * _L, _L)]
                sv = srcb[pl.ds(v * _L, _L)]

                def edge(l, _):
                    j = v * _L + l
                    ps = psb[j]
                    pd = pdb[j]
                    off = (ps - pd) * _INVR
                    r2 = jnp.sum(off * off)
                    t = 1.0 - r2
                    win = jnp.clip(t * t * t, 0.0, 1.0)
                    u = jnp.clip((off + 1.0) * 1.5, 0.0, 3.0)
                    i0 = jnp.minimum(u.astype(jnp.int32), 2)
                    f = u - i0.astype(jnp.float32)
                    fx = f[0]
                    fy = f[1]
                    fz = f[2]
                    cell0 = i0[0] * 16 + i0[1] * 4 + i0[2]
                    wx = jnp.where(mx, fx, 1.0 - fx)
                    wy = jnp.where(my, fy, 1.0 - fy)
                    wz = jnp.where(mz, fz, 1.0 - fz)
                    dr = jnp.sum(jnp.where(lane == l, dv, 0))
                    wine = jnp.where(dr < n_dst, win, 0.0)
                    w8 = jnp.where(lt8, wx * wy * wz * wine, 0.0)
                    srcj = jnp.sum(jnp.where(lane == l, sv, 0))
                    ivec = (jnp.where(lane == 8, srcj, 0)
                            + jnp.where(lane == 9, dr, 0)
                            + jnp.where(lane == 10, cell0, 0))
                    recb[j] = jnp.where(lt8, w8,
                                        plsc.bitcast(ivec, jnp.float32))
                    return 0

                lax.fori_loop(0, _L, edge, 0, unroll=2)
                return 0

            lax.fori_loop(0, _SCAN // _L, group, 0)
            pltpu.sync_copy(recb, rec_hbm.at[pl.ds(off0, _SCAN)])
            return 0

        lax.fori_loop(0, W // _SCAN, chunk, 0)

    return geom


def _make_scatter(e_pad, n_src, in_p, td):
    ce = {16: 128, 32: 128, 64: 96, 128: 32}[in_p]
    nt = _ROWS // td
    nt2 = nt // 2
    td_rows = td * 64
    esl = e_pad // 16
    assert esl % _SCAN == 0
    fr = td_rows // 16
    zr = 4096 // in_p
    nzc = fr // zr
    nv = in_p // _L
    sent = np.int32(e_pad - 1)
    mesh = plsc.VectorSubcoreMesh(core_axis_name="c", subcore_axis_name="s")

    @functools.partial(
        pl.kernel,
        out_type=jax.ShapeDtypeStruct((_ROWS * 64, in_p), jnp.float32),
        mesh=mesh,
        compiler_params=pltpu.CompilerParams(needs_layout_passes=False, use_tc_tiling_on_sc=False),
        scratch_types=[
            pltpu.VMEM_SHARED((td_rows, in_p), jnp.float32),
            pltpu.VMEM((esl + 128,), jnp.int32),
            pltpu.VMEM((esl,), jnp.int32),
            pltpu.VMEM((ce, 16), jnp.float32),
            pltpu.VMEM((ce,), jnp.int32),
            pltpu.VMEM((ce, in_p), jnp.float32),
            pltpu.VMEM((8, ce, in_p), jnp.float32),
            pltpu.VMEM((8, ce), jnp.int32),
            pltpu.VMEM((zr, in_p), jnp.float32),
            pltpu.SemaphoreType.DMA,
        ],
    )
    def scat(rec_hbm, dst_hbm, ft_hbm, acc_hbm,
             acc_sh, eids, dstsl, recb, srcb, featb, valb, idxb, zbuf, sem):
        c = lax.axis_index("c")
        s = lax.axis_index("s")
        lane = lax.iota(jnp.int32, 16)
        lt8 = lane < 8
        corner = (((lane >> 2) & 1) * 16 + ((lane >> 1) & 1) * 4 + (lane & 1))
        zv = jnp.zeros((16,), jnp.float32)

        def zrow(r, _):
            def zq(q, _):
                zbuf[r, pl.ds(q * _L, _L)] = zv
                return 0
            lax.fori_loop(0, nv, zq, 0)
            return 0

        lax.fori_loop(0, zr, zrow, 0)
        sbase = s * esl
        pltpu.sync_copy(dst_hbm.at[pl.ds(sbase, esl)], dstsl)

        def tile(i, _):
            t = i * 2 + c
            lo = t * td

            def zc(z, _):
                pltpu.sync_copy(zbuf, acc_sh.at[pl.ds(s * fr + z * zr, zr)])
                return 0

            lax.fori_loop(0, nzc, zc, 0)
            plsc.subcore_barrier()

            def scan(k, cnt):
                dv = dstsl[pl.ds(k * _L, _L)]
                m = (dv >= lo) & (dv < lo + td)
                ev = sbase + k * _L + lane
                plsc.store_compressed(eids.at[pl.ds(cnt, _L)], ev, mask=m)
                return cnt + plsc.all_reduce_population_count(m)[0]

            cnt = lax.fori_loop(0, esl // _L, scan, jnp.int32(0), unroll=2)
            sv = jnp.full((16,), sent, jnp.int32)
            for v in range(ce // _L):
                eids[pl.ds(cnt + v * _L, _L)] = sv

            def chunk(ch, _):
                pltpu.sync_copy(rec_hbm.at[eids.at[pl.ds(ch * ce, ce)]],
                                recb)
                for v in range(ce // _L):
                    rowi = v * _L + lane
                    coli = jnp.full((16,), 8, jnp.int32)
                    sw = plsc.load_gather(recb, [rowi, coli])
                    srcb[pl.ds(v * _L, _L)] = plsc.bitcast(sw, jnp.int32)
                pltpu.sync_copy(ft_hbm.at[srcb], featb)

                def edge(j, _):
                    rec = recb[j]
                    ivec = plsc.bitcast(rec, jnp.int32)
                    dstj = ivec[9]
                    cell0 = ivec[10]
                    rows8 = (dstj - lo) * 64 + cell0 + corner
                    rows8 = jnp.clip(rows8, 0, td_rows - 1)
                    plsc.store_scatter(
                        idxb, [lane, jnp.full((16,), j, jnp.int32)],
                        rows8, mask=lt8)
                    vq = [featb[j, pl.ds(q * _L, _L)] for q in range(nv)]
                    for cc in range(8):
                        wc = rec[cc]
                        for q in range(nv):
                            valb[cc, j, pl.ds(q * _L, _L)] = wc * vq[q]
                    return 0

                lax.fori_loop(0, ce, edge, 0, unroll=2)
                hs = [pltpu.async_copy(valb.at[cc],
                                       acc_sh.at[idxb.at[cc]],
                                       sem, add=True)
                      for cc in range(8)]
                for h in hs:
                    h.wait()
                return 0

            lax.fori_loop(0, (cnt + ce - 1) // ce, chunk, 0)
            plsc.subcore_barrier()
            pltpu.sync_copy(acc_sh.at[pl.ds(s * fr, fr)],
                            acc_hbm.at[pl.ds(t * td_rows + s * fr, fr)])
            return 0

        lax.fori_loop(0, nt2, tile, 0)

    return scat


def _mm_kernel(x_ref, w_ref, b_ref, o_ref):
    o_ref[...] = (jnp.dot(x_ref[...], w_ref[...],
                          preferred_element_type=jnp.float32)
                  + b_ref[...])


def _mm_dense_kernel(x_ref, w_ref, y_ref, wd_ref, b_ref, o_ref):
    o_ref[...] = (jnp.dot(x_ref[...], w_ref[...],
                          preferred_element_type=jnp.float32)
                  + jnp.dot(y_ref[...], wd_ref[...],
                            preferred_element_type=jnp.float32)
                  + b_ref[...])


_BR = 512


def _tc_matmul(x, w, b, y=None, wd=None):
    kd = x.shape[1]
    b2 = jnp.broadcast_to(b[None, :], (_BR, 128))
    grid = (_ROWS // _BR,)
    if y is None:
        return pl.pallas_call(
            _mm_kernel,
            grid=grid,
            in_specs=[
                pl.BlockSpec((_BR, kd), lambda i: (i, 0)),
                pl.BlockSpec((kd, 128), lambda i: (0, 0)),
                pl.BlockSpec((_BR, 128), lambda i: (0, 0)),
            ],
            out_specs=pl.BlockSpec((_BR, 128), lambda i: (i, 0)),
            out_shape=jax.ShapeDtypeStruct((_ROWS, 128), jnp.float32),
        )(x, w, b2)
    ip = y.shape[1]
    return pl.pallas_call(
        _mm_dense_kernel,
        grid=grid,
        in_specs=[
            pl.BlockSpec((_BR, kd), lambda i: (i, 0)),
            pl.BlockSpec((kd, 128), lambda i: (0, 0)),
            pl.BlockSpec((_BR, ip), lambda i: (i, 0)),
            pl.BlockSpec((ip, 128), lambda i: (0, 0)),
            pl.BlockSpec((_BR, 128), lambda i: (0, 0)),
        ],
        out_specs=pl.BlockSpec((_BR, 128), lambda i: (i, 0)),
        out_shape=jax.ShapeDtypeStruct((_ROWS, 128), jnp.float32),
    )(x, w, y, wd, b2)


def _pad_cols(a, w):
    return jnp.pad(a, ((0, 0), (0, w - a.shape[1])))


def _kmat(K, in_p):
    ci, co = K.shape[3], K.shape[4]
    km = K.reshape(64, ci, co)
    km = jnp.pad(km, ((0, 0), (0, in_p - ci), (0, 128 - co)))
    return km.reshape(64 * in_p, 128)


_TD_BY_INP = {16: 640, 32: 320, 64: 160, 128: 80}


def _bn(x, g, b):
    m = jnp.mean(x, axis=0)
    v = jnp.var(x, axis=0)
    return (x - m) / jnp.sqrt(v + 1e-5) * g + b


def kernel(pos, vel, box, box_feats, edge_fluid, edge_obs, params):
    p = params
    n = pos.shape[0]
    m = box.shape[0]
    e = edge_fluid.shape[1]
    eo = edge_obs.shape[1]
    e_pad = _ceil_to(e, 4096)
    eo_pad = _ceil_to(eo, 4096)

    pos16 = _pad_cols(pos, 16)
    box16 = _pad_cols(box, 16)

    def pad_edges(edges, epad):
        src = jnp.pad(edges[0], (0, epad - edges.shape[1]))
        dst = jnp.pad(edges[1], (0, epad - edges.shape[1]),
                      constant_values=_PAD_DST)
        return src, dst

    src_f, dst_f = pad_edges(edge_fluid, e_pad)
    src_o, dst_o = pad_edges(edge_obs, eo_pad)

    rec_f = _make_geometry(n, n, e_pad)(pos16, pos16, src_f, dst_f)
    rec_o = _make_geometry(m, n, eo_pad)(box16, pos16, src_o, dst_o)

    scat_cache = {}

    def conv(K, b, feats, rec, dst, epad, y=None, wd=None):
        ci = K.shape[3]
        in_p = max(16, _ceil_to(ci, 16))
        td = _TD_BY_INP[in_p]
        key = (epad, in_p)
        if key not in scat_cache:
            scat_cache[key] = _make_scatter(epad, feats.shape[0], in_p, td)
        ftp = _pad_cols(feats, in_p)
        acc = scat_cache[key](rec, dst, ftp)
        acc = acc.reshape(_ROWS, 64 * in_p)
        km = _kmat(K, in_p)
        if y is not None:
            ip = y.shape[1]
            yp = jnp.pad(y, ((0, _ROWS - n), (0, 0)))
            wdp = jnp.pad(wd, ((0, 0), (0, 128 - wd.shape[1])))
            out = _tc_matmul(acc, km, jnp.pad(b, (0, 128 - b.shape[0])),
                             yp, wdp)
        else:
            out = _tc_matmul(acc, km, jnp.pad(b, (0, 128 - b.shape[0])))
        return out[:n, :K.shape[4]]

    def fconv(K, b, feats, y=None, wd=None):
        return conv(K, b, feats, rec_f, dst_f, e_pad, y=y, wd=wd)

    feats = jnp.concatenate([jnp.ones((n, 1), pos.dtype), vel], axis=-1)
    xf = fconv(p['K0f'], p['b0f'], feats)
    xo = conv(p['K0o'], p['b0o'], box_feats, rec_o, dst_o, eo_pad)
    xd = feats @ p['W0'] + p['d0']

    x, y = xf + xo, xd
    xa = jnp.concatenate([x, y], axis=-1)
    xl = fconv(p['iaff_K1'], p['iaff_b1'], xa)
    xl = jax.nn.relu(_bn(xl, p['iaff_g1'], p['iaff_be1']))
    xl = _bn(fconv(p['iaff_K2'], p['iaff_b2'], xl), p['iaff_g2'], p['iaff_be2'])
    w1 = jax.nn.sigmoid(xl)
    xg = 2.0 * x * w1 + 2.0 * y * (1.0 - w1)
    t = fconv(p['iaff_K3'], p['iaff_b3'], xg)
    t = jax.nn.relu(_bn(t, p['iaff_g3'], p['iaff_be3']))
    t = _bn(fconv(p['iaff_K4'], p['iaff_b4'], t), p['iaff_g4'], p['iaff_be4'])
    w2 = jax.nn.sigmoid(t)
    x32 = 2.0 * x * w2 + 2.0 * y * (1.0 - w2)

    x = jnp.concatenate([x32, xd], axis=-1)
    for i in range(1, 5):
        inp = x if i == 1 else jax.nn.relu(x)
        y = fconv(p['K%d' % i], p['bk%d' % i], inp,
                  y=inp, wd=p['W%d' % i]) + p['d%d' % i]
        if y.shape[-1] == x.shape[-1]:
            y = y + x
        x = y
    return x * (1.0 / 128.0)
